# baseline (speedup 1.0000x reference)
import functools

import jax
import jax.numpy as jnp
from jax import lax
from jax.experimental import pallas as pl
from jax.experimental.pallas import tpu as pltpu
from jax.experimental.pallas import tpu_sc as plsc

N = 10000
E = 320000
D_IN = 256
LAT = 64
DEC = 128
HID = 128
NCLS = 16

NC = 2
NS = 16
NW = NC * NS
EPW = E // NW
NPAD = 10240
SLAB = NPAD // NS
WB = 32

_HIGH = jax.lax.Precision.HIGHEST


def _cov_body(x0_ref, x1_ref, c0_ref, s0_ref, c1_ref, s1_ref):
    @pl.when(pl.program_id(0) == 0)
    def _init():
        c0_ref[...] = jnp.zeros_like(c0_ref)
        s0_ref[...] = jnp.zeros_like(s0_ref)
        c1_ref[...] = jnp.zeros_like(c1_ref)
        s1_ref[...] = jnp.zeros_like(s1_ref)

    for x_ref, c_ref, s_ref in ((x0_ref, c0_ref, s0_ref),
                                (x1_ref, c1_ref, s1_ref)):
        x = x_ref[...]
        c_ref[...] += lax.dot_general(x, x, (((0,), (0,)), ((), ())),
                                      preferred_element_type=jnp.float32)
        s_ref[...] += jnp.sum(x, axis=0, keepdims=True)


def _cov(x0, x1):
    blk = 1000
    grid = N // blk
    return pl.pallas_call(
        _cov_body,
        grid=(grid,),
        in_specs=[pl.BlockSpec((blk, D_IN), lambda i: (i, 0)),
                  pl.BlockSpec((blk, D_IN), lambda i: (i, 0))],
        out_specs=[pl.BlockSpec((D_IN, D_IN), lambda i: (0, 0)),
                   pl.BlockSpec((1, D_IN), lambda i: (0, 0)),
                   pl.BlockSpec((D_IN, D_IN), lambda i: (0, 0)),
                   pl.BlockSpec((1, D_IN), lambda i: (0, 0))],
        out_shape=[jax.ShapeDtypeStruct((D_IN, D_IN), jnp.float32),
                   jax.ShapeDtypeStruct((1, D_IN), jnp.float32),
                   jax.ShapeDtypeStruct((D_IN, D_IN), jnp.float32),
                   jax.ShapeDtypeStruct((1, D_IN), jnp.float32)],
    )(x0, x1)


def _fold_body(c0_ref, s0_ref, c1_ref, s1_ref,
               w1_0, b1_0, g1_0, be1_0, w2_0, b2_0, g2_0, be2_0, dw_0, db_0,
               w1_1, b1_1, g1_1, be1_1, w2_1, b2_1, g2_1, be2_1, dw_1, db_1,
               gw0_ref, g0_ref, g1o_ref, gb_ref):
    def mm(a, b):
        return jnp.dot(a, b, preferred_element_type=jnp.float32,
                       precision=_HIGH)

    gw0 = gw0_ref[...]
    bds = None
    for (c_ref, s_ref, W1, b1, g1, be1, W2, b2, g2, be2, dW, db, g_ref) in (
            (c0_ref, s0_ref, w1_0, b1_0, g1_0, be1_0, w2_0, b2_0, g2_0, be2_0,
             dw_0, db_0, g0_ref),
            (c1_ref, s1_ref, w1_1, b1_1, g1_1, be1_1, w2_1, b2_1, g2_1, be2_1,
             dw_1, db_1, g1o_ref)):
        mu = s_ref[...] / N
        cov = c_ref[...] / N - lax.dot_general(
            mu, mu, (((0,), (0,)), ((), ())),
            preferred_element_type=jnp.float32, precision=_HIGH)
        W1v = W1[...]
        mu1 = mm(mu, W1v) + b1[...]
        var1 = jnp.sum(W1v * mm(cov, W1v), axis=0, keepdims=True)
        a1 = g1[...] * lax.rsqrt(var1 + 1e-5)
        d1 = (b1[...] - mu1) * a1 + be1[...]
        W2p = mm(W1v * a1, W2[...])
        b2p = mm(d1, W2[...]) + b2[...]
        mu2 = mm(mu, W2p) + b2p
        var2 = jnp.sum(W2p * mm(cov, W2p), axis=0, keepdims=True)
        a2 = g2[...] * lax.rsqrt(var2 + 1e-5)
        d2 = (b2p - mu2) * a2 + be2[...]
        Wd = mm(W2p * a2, dW[...])
        bd = mm(d2, dW[...]) + db[...]
        g_ref[...] = mm(Wd, gw0) * 0.5
        bds = bd if bds is None else bds + bd
    gb_ref[...] = mm(bds * 0.5, gw0)


def _fold(c0, s0, c1, s1, p):
    args = [c0, s0, c1, s1]
    for m in range(2):
        args += [p[f'enc{m}_W1'], p[f'enc{m}_b1'][None, :],
                 p[f'enc{m}_g1'][None, :], p[f'enc{m}_be1'][None, :],
                 p[f'enc{m}_W2'], p[f'enc{m}_b2'][None, :],
                 p[f'enc{m}_g2'][None, :], p[f'enc{m}_be2'][None, :],
                 p[f'dec{m}_W'], p[f'dec{m}_b'][None, :]]
    args.append(p['gcn_W0'])
    return pl.pallas_call(
        _fold_body,
        out_shape=[jax.ShapeDtypeStruct((D_IN, HID), jnp.float32),
                   jax.ShapeDtypeStruct((D_IN, HID), jnp.float32),
                   jax.ShapeDtypeStruct((1, HID), jnp.float32)],
    )(*args)


def _deg_body(src_ref, dst_ref, hs_ref, hd_ref, idx_v, hist_v):
    cid = lax.axis_index("c")
    sid = lax.axis_index("s")
    wid = cid * NS + sid
    ones = jnp.full((16,), 1.0, jnp.float32)
    zeros = jnp.zeros((16,), jnp.float32)
    for e_ref, h_ref in ((src_ref, hs_ref), (dst_ref, hd_ref)):
        def zero_step(i, _):
            hist_v[pl.ds(i * 16, 16)] = zeros
            return _
        lax.fori_loop(0, N // 16, zero_step, None)
        pltpu.sync_copy(e_ref.at[pl.ds(wid * EPW, EPW)], idx_v)

        def add_step(i, _):
            idx = idx_v[pl.ds(i * 16, 16)]
            plsc.addupdate_scatter(hist_v, [idx], ones)
            return _
        lax.fori_loop(0, EPW // 16, add_step, None)
        pltpu.sync_copy(hist_v, h_ref.at[wid])


def _degrees(src, dst):
    k = pl.kernel(
        _deg_body,
        out_type=[jax.ShapeDtypeStruct((NW, N), jnp.float32),
                  jax.ShapeDtypeStruct((NW, N), jnp.float32)],
        mesh=plsc.VectorSubcoreMesh(core_axis_name="c", subcore_axis_name="s"),
        scratch_types=[pltpu.VMEM((EPW,), jnp.int32),
                       pltpu.VMEM((N,), jnp.float32)],
        compiler_params=pltpu.CompilerParams(needs_layout_passes=False),
    )
    return k(src, dst)


def _degfin_body(hs_ref, hd_ref, do_ref, di_ref):
    s = jnp.sum(hs_ref[...], axis=0, keepdims=True)
    do_ref[...] = lax.rsqrt(jnp.maximum(s, 1.0))
    d = jnp.sum(hd_ref[...], axis=0, keepdims=True)
    di_ref[...] = lax.rsqrt(jnp.maximum(d, 1.0))


def _degfin(hs, hd):
    return pl.pallas_call(
        _degfin_body,
        out_shape=[jax.ShapeDtypeStruct((1, N), jnp.float32),
                   jax.ShapeDtypeStruct((1, N), jnp.float32)],
    )(hs, hd)


def _z_body(x0_ref, x1_ref, g0_ref, g1_ref, gb_ref, do_ref, z_ref):
    z = (jnp.dot(x0_ref[...], g0_ref[...], preferred_element_type=jnp.float32)
         + jnp.dot(x1_ref[...], g1_ref[...],
                   preferred_element_type=jnp.float32)
         + gb_ref[...])
    z_ref[...] = z * do_ref[...]


def _z_kernel(x0, x1, g0, g1, gb, dof):
    blk = 1000
    return pl.pallas_call(
        _z_body,
        grid=(N // blk,),
        in_specs=[pl.BlockSpec((blk, D_IN), lambda i: (i, 0)),
                  pl.BlockSpec((blk, D_IN), lambda i: (i, 0)),
                  pl.BlockSpec((D_IN, HID), lambda i: (0, 0)),
                  pl.BlockSpec((D_IN, HID), lambda i: (0, 0)),
                  pl.BlockSpec((1, HID), lambda i: (0, 0)),
                  pl.BlockSpec((blk, 1), lambda i: (i, 0))],
        out_specs=pl.BlockSpec((blk, HID), lambda i: (i, 0)),
        out_shape=jax.ShapeDtypeStruct((N, HID), jnp.float32),
    )(x0, x1, g0, g1, gb, dof)


def _make_agg(d, kc, nstg):
    nchunk = EPW // kc
    npair = nstg // 2

    def body(tab_ref, srcm_ref, dstm_ref, out_ref, sidx_v, didx_v, rows_a,
             rows_b, wbuf_v, acc_sh, sem_a, sem_b):
        cid = lax.axis_index("c")
        sid = lax.axis_index("s")
        wid = cid * NS + sid
        zeros = jnp.zeros((16,), jnp.float32)

        def zrow(i, _):
            for j in range(d // 16):
                wbuf_v[i, pl.ds(j * 16, 16)] = zeros
            return _
        lax.fori_loop(0, WB, zrow, None)
        for k in range(SLAB // WB):
            pltpu.sync_copy(wbuf_v, acc_sh.at[pl.ds(sid * SLAB + k * WB, WB)])
        plsc.subcore_barrier()

        def pair(j, _):
            for rows_v, sem, par in ((rows_a, sem_a, 0), (rows_b, sem_b, 1)):
                c = 2 * j + par
                pltpu.make_async_copy(tab_ref.at[sidx_v.at[c]], rows_v,
                                      sem).wait()
                pltpu.sync_copy(rows_v, acc_sh.at[didx_v.at[c]], add=True)

                @pl.when(j + 1 < npair)
                def _refill():
                    pltpu.async_copy(tab_ref.at[sidx_v.at[c + 2]], rows_v,
                                     sem)
            return _

        for s in range(nchunk // nstg):
            base = wid * nchunk + s * nstg
            pltpu.sync_copy(srcm_ref.at[pl.ds(base, nstg)], sidx_v)
            pltpu.sync_copy(dstm_ref.at[pl.ds(base, nstg)], didx_v)
            pltpu.async_copy(tab_ref.at[sidx_v.at[0]], rows_a, sem_a)
            pltpu.async_copy(tab_ref.at[sidx_v.at[1]], rows_b, sem_b)
            lax.fori_loop(0, npair, pair, None)
        plsc.subcore_barrier()

        for k in range(SLAB // WB):
            off = sid * SLAB + k * WB
            pltpu.sync_copy(acc_sh.at[pl.ds(off, WB)], wbuf_v)
            pltpu.sync_copy(wbuf_v, out_ref.at[cid, pl.ds(off, WB)])

    def agg(table, srcm, dstm):
        k = pl.kernel(
            body,
            out_type=jax.ShapeDtypeStruct((NC, NPAD, d), jnp.float32),
            mesh=plsc.VectorSubcoreMesh(core_axis_name="c",
                                        subcore_axis_name="s"),
            scratch_types=[pltpu.VMEM((nstg, kc), jnp.int32),
                           pltpu.VMEM((nstg, kc), jnp.int32),
                           pltpu.VMEM((kc, d), jnp.float32),
                           pltpu.VMEM((kc, d), jnp.float32),
                           pltpu.VMEM((WB, d), jnp.float32),
                           pltpu.VMEM_SHARED((NPAD, d), jnp.float32),
                           pltpu.SemaphoreType.DMA,
                           pltpu.SemaphoreType.DMA],
            compiler_params=pltpu.CompilerParams(
                needs_layout_passes=False,
                use_tc_tiling_on_sc=(d % 128 == 0)),
        )
        return k(table, srcm, dstm)

    return agg


def _make_agg_ring(d, kc, nstg):
    nchunk = EPW // kc

    def body(tab_ref, srcm_ref, dstm_ref, out_ref, sidx_v, didx_v, b0, b1,
             b2, b3, wbuf_v, acc_sh, sg0, sg1, sg2, sg3, ss0, ss1, ss2, ss3):
        cid = lax.axis_index("c")
        sid = lax.axis_index("s")
        wid = cid * NS + sid
        zeros = jnp.zeros((16,), jnp.float32)
        bufs = (b0, b1, b2, b3)
        sgs = (sg0, sg1, sg2, sg3)
        sss = (ss0, ss1, ss2, ss3)

        def zrow(i, _):
            for j in range(d // 16):
                wbuf_v[i, pl.ds(j * 16, 16)] = zeros
            return _
        lax.fori_loop(0, WB, zrow, None)
        for k in range(SLAB // WB):
            pltpu.sync_copy(wbuf_v, acc_sh.at[pl.ds(sid * SLAB + k * WB, WB)])
        plsc.subcore_barrier()

        def quad(j, _):
            for q in range(4):
                c = 4 * j + q
                r = (q + 2) % 4
                pltpu.make_async_copy(tab_ref.at[sidx_v.at[c]], bufs[q],
                                      sgs[q]).wait()
                pltpu.async_copy(bufs[q], acc_sh.at[didx_v.at[c]], sss[q],
                                 add=True)

                @pl.when((c >= 2) & (c + 2 < nstg))
                def _wait_sc():
                    pltpu.make_async_copy(bufs[r],
                                          acc_sh.at[didx_v.at[c - 2]],
                                          sss[r]).wait()

                @pl.when(c + 2 < nstg)
                def _refill():
                    pltpu.async_copy(tab_ref.at[sidx_v.at[c + 2]], bufs[r],
                                     sgs[r])
            return _

        for s in range(nchunk // nstg):
            base = wid * nchunk + s * nstg
            pltpu.sync_copy(srcm_ref.at[pl.ds(base, nstg)], sidx_v)
            pltpu.sync_copy(dstm_ref.at[pl.ds(base, nstg)], didx_v)
            pltpu.async_copy(tab_ref.at[sidx_v.at[0]], b0, sg0)
            pltpu.async_copy(tab_ref.at[sidx_v.at[1]], b1, sg1)
            lax.fori_loop(0, nstg // 4, quad, None)
            for q in range(4):
                pltpu.make_async_copy(bufs[q],
                                      acc_sh.at[didx_v.at[nstg - 4 + q]],
                                      sss[q]).wait()
        plsc.subcore_barrier()

        for k in range(SLAB // WB):
            off = sid * SLAB + k * WB
            pltpu.sync_copy(acc_sh.at[pl.ds(off, WB)], wbuf_v)
            pltpu.sync_copy(wbuf_v, out_ref.at[cid, pl.ds(off, WB)])

    def agg(table, srcm, dstm):
        k = pl.kernel(
            body,
            out_type=jax.ShapeDtypeStruct((NC, NPAD, d), jnp.float32),
            mesh=plsc.VectorSubcoreMesh(core_axis_name="c",
                                        subcore_axis_name="s"),
            scratch_types=[pltpu.VMEM((nstg, kc), jnp.int32),
                           pltpu.VMEM((nstg, kc), jnp.int32),
                           pltpu.VMEM((kc, d), jnp.float32),
                           pltpu.VMEM((kc, d), jnp.float32),
                           pltpu.VMEM((kc, d), jnp.float32),
                           pltpu.VMEM((kc, d), jnp.float32),
                           pltpu.VMEM((WB, d), jnp.float32),
                           pltpu.VMEM_SHARED((NPAD, d), jnp.float32)]
            + [pltpu.SemaphoreType.DMA] * 8,
            compiler_params=pltpu.CompilerParams(
                needs_layout_passes=False,
                use_tc_tiling_on_sc=(d % 128 == 0)),
        )
        return k(table, srcm, dstm)

    return agg


KC128 = 25
KC16 = 125
_agg128 = _make_agg_ring(HID, KC128, 40)
_agg16 = _make_agg_ring(NCLS, KC16, 16)


def _mid_body(ap_ref, di_ref, do_ref, b0_ref, w1_ref, g_ref):
    s = ap_ref[0] + ap_ref[1]
    h0 = jnp.maximum(s * di_ref[...] + b0_ref[...], 0.0)
    g = jnp.dot(h0, w1_ref[...], preferred_element_type=jnp.float32,
                precision=_HIGH)
    g_ref[...] = g * do_ref[...]


def _mid(aggp, dif, dof, b0, w1):
    blk = 1000
    return pl.pallas_call(
        _mid_body,
        grid=(N // blk,),
        in_specs=[pl.BlockSpec((NC, blk, HID), lambda i: (0, i, 0)),
                  pl.BlockSpec((blk, 1), lambda i: (i, 0)),
                  pl.BlockSpec((blk, 1), lambda i: (i, 0)),
                  pl.BlockSpec((1, HID), lambda i: (0, 0)),
                  pl.BlockSpec((HID, NCLS), lambda i: (0, 0))],
        out_specs=pl.BlockSpec((blk, NCLS), lambda i: (i, 0)),
        out_shape=jax.ShapeDtypeStruct((N, NCLS), jnp.float32),
    )(aggp, dif, dof, b0, w1)


def _fin_body(ap_ref, di_ref, b1_ref, o_ref):
    o_ref[...] = (ap_ref[0] + ap_ref[1]) * di_ref[...] + b1_ref[...]


def _final(aggp2, dif, b1):
    blk = 1000
    return pl.pallas_call(
        _fin_body,
        grid=(N // blk,),
        in_specs=[pl.BlockSpec((NC, blk, NCLS), lambda i: (0, i, 0)),
                  pl.BlockSpec((blk, 1), lambda i: (i, 0)),
                  pl.BlockSpec((1, NCLS), lambda i: (0, 0))],
        out_specs=pl.BlockSpec((blk, NCLS), lambda i: (i, 0)),
        out_shape=jax.ShapeDtypeStruct((N, NCLS), jnp.float32),
    )(aggp2, dif, b1)


def kernel(x0, x1, params, edge_index):
    src = edge_index[0]
    dst = edge_index[1]
    srcm128 = src.reshape(E // KC128, KC128)
    dstm128 = dst.reshape(E // KC128, KC128)
    srcm16 = src.reshape(E // KC16, KC16)
    dstm16 = dst.reshape(E // KC16, KC16)

    hs, hd = _degrees(src, dst)
    dof2, dif2 = _degfin(hs, hd)
    dof = dof2.reshape(N, 1)
    dif = dif2.reshape(N, 1)

    c0, s0, c1, s1 = _cov(x0, x1)
    g0, g1, gb = _fold(c0, s0, c1, s1, params)
    z = _z_kernel(x0, x1, g0, g1, gb, dof)

    aggp = _agg128(z, srcm128, dstm128)
    g = _mid(aggp, dif, dof, params['gcn_b0'][None, :], params['gcn_W1'])
    aggp2 = _agg16(g, srcm16, dstm16)
    return _final(aggp2, dif, params['gcn_b1'][None, :])


# speedup vs baseline: 1.1546x; 1.1546x over previous
import functools

import jax
import jax.numpy as jnp
from jax import lax
from jax.experimental import pallas as pl
from jax.experimental.pallas import tpu as pltpu
from jax.experimental.pallas import tpu_sc as plsc

N = 10000
E = 320000
D_IN = 256
LAT = 64
DEC = 128
HID = 128
NCLS = 16

NC = 2
NS = 16
NW = NC * NS
EPW = E // NW
NPAD = 10240
SLAB = NPAD // NS
WB = 32

_HIGH = jax.lax.Precision.HIGHEST


def _cov_body(x0_ref, x1_ref, c0_ref, s0_ref, c1_ref, s1_ref):
    @pl.when(pl.program_id(0) == 0)
    def _init():
        c0_ref[...] = jnp.zeros_like(c0_ref)
        s0_ref[...] = jnp.zeros_like(s0_ref)
        c1_ref[...] = jnp.zeros_like(c1_ref)
        s1_ref[...] = jnp.zeros_like(s1_ref)

    for x_ref, c_ref, s_ref in ((x0_ref, c0_ref, s0_ref),
                                (x1_ref, c1_ref, s1_ref)):
        x = x_ref[...]
        c_ref[...] += lax.dot_general(x, x, (((0,), (0,)), ((), ())),
                                      preferred_element_type=jnp.float32)
        s_ref[...] += jnp.sum(x, axis=0, keepdims=True)


def _cov(x0, x1):
    blk = 1000
    grid = N // blk
    return pl.pallas_call(
        _cov_body,
        grid=(grid,),
        in_specs=[pl.BlockSpec((blk, D_IN), lambda i: (i, 0)),
                  pl.BlockSpec((blk, D_IN), lambda i: (i, 0))],
        out_specs=[pl.BlockSpec((D_IN, D_IN), lambda i: (0, 0)),
                   pl.BlockSpec((1, D_IN), lambda i: (0, 0)),
                   pl.BlockSpec((D_IN, D_IN), lambda i: (0, 0)),
                   pl.BlockSpec((1, D_IN), lambda i: (0, 0))],
        out_shape=[jax.ShapeDtypeStruct((D_IN, D_IN), jnp.float32),
                   jax.ShapeDtypeStruct((1, D_IN), jnp.float32),
                   jax.ShapeDtypeStruct((D_IN, D_IN), jnp.float32),
                   jax.ShapeDtypeStruct((1, D_IN), jnp.float32)],
    )(x0, x1)


def _fold_body(c0_ref, s0_ref, c1_ref, s1_ref,
               w1_0, b1_0, g1_0, be1_0, w2_0, b2_0, g2_0, be2_0, dw_0, db_0,
               w1_1, b1_1, g1_1, be1_1, w2_1, b2_1, g2_1, be2_1, dw_1, db_1,
               gw0_ref, g0_ref, g1o_ref, gb_ref):
    def mm(a, b):
        return jnp.dot(a, b, preferred_element_type=jnp.float32,
                       precision=_HIGH)

    gw0 = gw0_ref[...]
    bds = None
    for (c_ref, s_ref, W1, b1, g1, be1, W2, b2, g2, be2, dW, db, g_ref) in (
            (c0_ref, s0_ref, w1_0, b1_0, g1_0, be1_0, w2_0, b2_0, g2_0, be2_0,
             dw_0, db_0, g0_ref),
            (c1_ref, s1_ref, w1_1, b1_1, g1_1, be1_1, w2_1, b2_1, g2_1, be2_1,
             dw_1, db_1, g1o_ref)):
        mu = s_ref[...] / N
        cov = c_ref[...] / N - lax.dot_general(
            mu, mu, (((0,), (0,)), ((), ())),
            preferred_element_type=jnp.float32, precision=_HIGH)
        W1v = W1[...]
        mu1 = mm(mu, W1v) + b1[...]
        var1 = jnp.sum(W1v * mm(cov, W1v), axis=0, keepdims=True)
        a1 = g1[...] * lax.rsqrt(var1 + 1e-5)
        d1 = (b1[...] - mu1) * a1 + be1[...]
        W2p = mm(W1v * a1, W2[...])
        b2p = mm(d1, W2[...]) + b2[...]
        mu2 = mm(mu, W2p) + b2p
        var2 = jnp.sum(W2p * mm(cov, W2p), axis=0, keepdims=True)
        a2 = g2[...] * lax.rsqrt(var2 + 1e-5)
        d2 = (b2p - mu2) * a2 + be2[...]
        Wd = mm(W2p * a2, dW[...])
        bd = mm(d2, dW[...]) + db[...]
        g_ref[...] = mm(Wd, gw0) * 0.5
        bds = bd if bds is None else bds + bd
    gb_ref[...] = mm(bds * 0.5, gw0)


def _fold(c0, s0, c1, s1, p):
    args = [c0, s0, c1, s1]
    for m in range(2):
        args += [p[f'enc{m}_W1'], p[f'enc{m}_b1'][None, :],
                 p[f'enc{m}_g1'][None, :], p[f'enc{m}_be1'][None, :],
                 p[f'enc{m}_W2'], p[f'enc{m}_b2'][None, :],
                 p[f'enc{m}_g2'][None, :], p[f'enc{m}_be2'][None, :],
                 p[f'dec{m}_W'], p[f'dec{m}_b'][None, :]]
    args.append(p['gcn_W0'])
    return pl.pallas_call(
        _fold_body,
        out_shape=[jax.ShapeDtypeStruct((D_IN, HID), jnp.float32),
                   jax.ShapeDtypeStruct((D_IN, HID), jnp.float32),
                   jax.ShapeDtypeStruct((1, HID), jnp.float32)],
    )(*args)


def _deg_body(src_ref, dst_ref, hs_ref, hd_ref, idx_v, hist_v):
    cid = lax.axis_index("c")
    sid = lax.axis_index("s")
    wid = cid * NS + sid
    ones = jnp.full((16,), 1.0, jnp.float32)
    zeros = jnp.zeros((16,), jnp.float32)
    for e_ref, h_ref in ((src_ref, hs_ref), (dst_ref, hd_ref)):
        def zero_step(i, _):
            hist_v[pl.ds(i * 16, 16)] = zeros
            return _
        lax.fori_loop(0, N // 16, zero_step, None)
        pltpu.sync_copy(e_ref.at[pl.ds(wid * EPW, EPW)], idx_v)

        def add_step(i, _):
            idx = idx_v[pl.ds(i * 16, 16)]
            plsc.addupdate_scatter(hist_v, [idx], ones)
            return _
        lax.fori_loop(0, EPW // 16, add_step, None)
        pltpu.sync_copy(hist_v, h_ref.at[wid])


def _degrees(src, dst):
    k = pl.kernel(
        _deg_body,
        out_type=[jax.ShapeDtypeStruct((NW, N), jnp.float32),
                  jax.ShapeDtypeStruct((NW, N), jnp.float32)],
        mesh=plsc.VectorSubcoreMesh(core_axis_name="c", subcore_axis_name="s"),
        scratch_types=[pltpu.VMEM((EPW,), jnp.int32),
                       pltpu.VMEM((N,), jnp.float32)],
        compiler_params=pltpu.CompilerParams(needs_layout_passes=False),
    )
    return k(src, dst)


def _degfin_body(hs_ref, hd_ref, do_ref, di_ref):
    s = jnp.sum(hs_ref[...], axis=0, keepdims=True)
    do_ref[...] = lax.rsqrt(jnp.maximum(s, 1.0))
    d = jnp.sum(hd_ref[...], axis=0, keepdims=True)
    di_ref[...] = lax.rsqrt(jnp.maximum(d, 1.0))


def _degfin(hs, hd):
    return pl.pallas_call(
        _degfin_body,
        out_shape=[jax.ShapeDtypeStruct((1, N), jnp.float32),
                   jax.ShapeDtypeStruct((1, N), jnp.float32)],
    )(hs, hd)


def _z_body(x0_ref, x1_ref, g0_ref, g1_ref, gb_ref, do_ref, z_ref):
    z = (jnp.dot(x0_ref[...], g0_ref[...], preferred_element_type=jnp.float32)
         + jnp.dot(x1_ref[...], g1_ref[...],
                   preferred_element_type=jnp.float32)
         + gb_ref[...])
    z_ref[...] = z * do_ref[...]


def _z_kernel(x0, x1, g0, g1, gb, dof):
    blk = 1000
    return pl.pallas_call(
        _z_body,
        grid=(N // blk,),
        in_specs=[pl.BlockSpec((blk, D_IN), lambda i: (i, 0)),
                  pl.BlockSpec((blk, D_IN), lambda i: (i, 0)),
                  pl.BlockSpec((D_IN, HID), lambda i: (0, 0)),
                  pl.BlockSpec((D_IN, HID), lambda i: (0, 0)),
                  pl.BlockSpec((1, HID), lambda i: (0, 0)),
                  pl.BlockSpec((blk, 1), lambda i: (i, 0))],
        out_specs=pl.BlockSpec((blk, HID), lambda i: (i, 0)),
        out_shape=jax.ShapeDtypeStruct((N, HID), jnp.float32),
    )(x0, x1, g0, g1, gb, dof)


def _make_agg(d, kc, nstg):
    nchunk = EPW // kc
    npair = nstg // 2

    def body(tab_ref, srcm_ref, dstm_ref, out_ref, sidx_v, didx_v, rows_a,
             rows_b, wbuf_v, acc_sh, sem_a, sem_b):
        cid = lax.axis_index("c")
        sid = lax.axis_index("s")
        wid = cid * NS + sid
        zeros = jnp.zeros((16,), jnp.float32)

        def zrow(i, _):
            for j in range(d // 16):
                wbuf_v[i, pl.ds(j * 16, 16)] = zeros
            return _
        lax.fori_loop(0, WB, zrow, None)
        for k in range(SLAB // WB):
            pltpu.sync_copy(wbuf_v, acc_sh.at[pl.ds(sid * SLAB + k * WB, WB)])
        plsc.subcore_barrier()

        def pair(j, _):
            for rows_v, sem, par in ((rows_a, sem_a, 0), (rows_b, sem_b, 1)):
                c = 2 * j + par
                pltpu.make_async_copy(tab_ref.at[sidx_v.at[c]], rows_v,
                                      sem).wait()
                pltpu.sync_copy(rows_v, acc_sh.at[didx_v.at[c]], add=True)

                @pl.when(j + 1 < npair)
                def _refill():
                    pltpu.async_copy(tab_ref.at[sidx_v.at[c + 2]], rows_v,
                                     sem)
            return _

        for s in range(nchunk // nstg):
            base = wid * nchunk + s * nstg
            pltpu.sync_copy(srcm_ref.at[pl.ds(base, nstg)], sidx_v)
            pltpu.sync_copy(dstm_ref.at[pl.ds(base, nstg)], didx_v)
            pltpu.async_copy(tab_ref.at[sidx_v.at[0]], rows_a, sem_a)
            pltpu.async_copy(tab_ref.at[sidx_v.at[1]], rows_b, sem_b)
            lax.fori_loop(0, npair, pair, None)
        plsc.subcore_barrier()

        for k in range(SLAB // WB):
            off = sid * SLAB + k * WB
            pltpu.sync_copy(acc_sh.at[pl.ds(off, WB)], wbuf_v)
            pltpu.sync_copy(wbuf_v, out_ref.at[cid, pl.ds(off, WB)])

    def agg(table, srcm, dstm):
        k = pl.kernel(
            body,
            out_type=jax.ShapeDtypeStruct((NC, NPAD, d), jnp.float32),
            mesh=plsc.VectorSubcoreMesh(core_axis_name="c",
                                        subcore_axis_name="s"),
            scratch_types=[pltpu.VMEM((nstg, kc), jnp.int32),
                           pltpu.VMEM((nstg, kc), jnp.int32),
                           pltpu.VMEM((kc, d), jnp.float32),
                           pltpu.VMEM((kc, d), jnp.float32),
                           pltpu.VMEM((WB, d), jnp.float32),
                           pltpu.VMEM_SHARED((NPAD, d), jnp.float32),
                           pltpu.SemaphoreType.DMA,
                           pltpu.SemaphoreType.DMA],
            compiler_params=pltpu.CompilerParams(
                needs_layout_passes=False,
                use_tc_tiling_on_sc=(d % 128 == 0)),
        )
        return k(table, srcm, dstm)

    return agg


def _make_agg_ring(d, kc, nstg):
    nchunk = EPW // kc

    def body(tab_ref, srcm_ref, dstm_ref, out_ref, sidx_v, didx_v, b0, b1,
             b2, b3, wbuf_v, acc_sh, sg0, sg1, sg2, sg3, ss0, ss1, ss2, ss3):
        cid = lax.axis_index("c")
        sid = lax.axis_index("s")
        wid = cid * NS + sid
        zeros = jnp.zeros((16,), jnp.float32)
        bufs = (b0, b1, b2, b3)
        sgs = (sg0, sg1, sg2, sg3)
        sss = (ss0, ss1, ss2, ss3)

        def zrow(i, _):
            for j in range(d // 16):
                wbuf_v[i, pl.ds(j * 16, 16)] = zeros
            return _
        lax.fori_loop(0, WB, zrow, None)
        for k in range(SLAB // WB):
            pltpu.sync_copy(wbuf_v, acc_sh.at[pl.ds(sid * SLAB + k * WB, WB)])
        plsc.subcore_barrier()

        def quad(j, _):
            for q in range(4):
                c = 4 * j + q
                r = (q + 2) % 4
                pltpu.make_async_copy(tab_ref.at[sidx_v.at[c]], bufs[q],
                                      sgs[q]).wait()
                pltpu.async_copy(bufs[q], acc_sh.at[didx_v.at[c]], sss[q],
                                 add=True)

                @pl.when((c >= 2) & (c + 2 < nstg))
                def _wait_sc():
                    pltpu.make_async_copy(bufs[r],
                                          acc_sh.at[didx_v.at[c - 2]],
                                          sss[r]).wait()

                @pl.when(c + 2 < nstg)
                def _refill():
                    pltpu.async_copy(tab_ref.at[sidx_v.at[c + 2]], bufs[r],
                                     sgs[r])
            return _

        for s in range(nchunk // nstg):
            base = wid * nchunk + s * nstg
            pltpu.sync_copy(srcm_ref.at[pl.ds(base, nstg)], sidx_v)
            pltpu.sync_copy(dstm_ref.at[pl.ds(base, nstg)], didx_v)
            pltpu.async_copy(tab_ref.at[sidx_v.at[0]], b0, sg0)
            pltpu.async_copy(tab_ref.at[sidx_v.at[1]], b1, sg1)
            lax.fori_loop(0, nstg // 4, quad, None)
            for q in range(4):
                pltpu.make_async_copy(bufs[q],
                                      acc_sh.at[didx_v.at[nstg - 4 + q]],
                                      sss[q]).wait()
        plsc.subcore_barrier()

        for k in range(SLAB // WB):
            off = sid * SLAB + k * WB
            pltpu.sync_copy(acc_sh.at[pl.ds(off, WB)], wbuf_v)
            pltpu.sync_copy(wbuf_v, out_ref.at[cid, pl.ds(off, WB)])

    def agg(table, srcm, dstm):
        k = pl.kernel(
            body,
            out_type=jax.ShapeDtypeStruct((NC, NPAD, d), jnp.float32),
            mesh=plsc.VectorSubcoreMesh(core_axis_name="c",
                                        subcore_axis_name="s"),
            scratch_types=[pltpu.VMEM((nstg, kc), jnp.int32),
                           pltpu.VMEM((nstg, kc), jnp.int32),
                           pltpu.VMEM((kc, d), jnp.float32),
                           pltpu.VMEM((kc, d), jnp.float32),
                           pltpu.VMEM((kc, d), jnp.float32),
                           pltpu.VMEM((kc, d), jnp.float32),
                           pltpu.VMEM((WB, d), jnp.float32),
                           pltpu.VMEM_SHARED((NPAD, d), jnp.float32)]
            + [pltpu.SemaphoreType.DMA] * 8,
            compiler_params=pltpu.CompilerParams(
                needs_layout_passes=False,
                use_tc_tiling_on_sc=(d % 128 == 0)),
        )
        return k(table, srcm, dstm)

    return agg


KC128 = 50
KC16 = 125
_agg128 = _make_agg(HID, KC128, 40)
_agg16 = _make_agg_ring(NCLS, KC16, 16)


def _mid_body(ap_ref, di_ref, do_ref, b0_ref, w1_ref, g_ref):
    s = ap_ref[0] + ap_ref[1]
    h0 = jnp.maximum(s * di_ref[...] + b0_ref[...], 0.0)
    g = jnp.dot(h0, w1_ref[...], preferred_element_type=jnp.float32,
                precision=_HIGH)
    g_ref[...] = g * do_ref[...]


def _mid(aggp, dif, dof, b0, w1):
    blk = 1000
    return pl.pallas_call(
        _mid_body,
        grid=(N // blk,),
        in_specs=[pl.BlockSpec((NC, blk, HID), lambda i: (0, i, 0)),
                  pl.BlockSpec((blk, 1), lambda i: (i, 0)),
                  pl.BlockSpec((blk, 1), lambda i: (i, 0)),
                  pl.BlockSpec((1, HID), lambda i: (0, 0)),
                  pl.BlockSpec((HID, NCLS), lambda i: (0, 0))],
        out_specs=pl.BlockSpec((blk, NCLS), lambda i: (i, 0)),
        out_shape=jax.ShapeDtypeStruct((N, NCLS), jnp.float32),
    )(aggp, dif, dof, b0, w1)


def _fin_body(ap_ref, di_ref, b1_ref, o_ref):
    o_ref[...] = (ap_ref[0] + ap_ref[1]) * di_ref[...] + b1_ref[...]


def _final(aggp2, dif, b1):
    blk = 1000
    return pl.pallas_call(
        _fin_body,
        grid=(N // blk,),
        in_specs=[pl.BlockSpec((NC, blk, NCLS), lambda i: (0, i, 0)),
                  pl.BlockSpec((blk, 1), lambda i: (i, 0)),
                  pl.BlockSpec((1, NCLS), lambda i: (0, 0))],
        out_specs=pl.BlockSpec((blk, NCLS), lambda i: (i, 0)),
        out_shape=jax.ShapeDtypeStruct((N, NCLS), jnp.float32),
    )(aggp2, dif, b1)


def kernel(x0, x1, params, edge_index):
    src = edge_index[0]
    dst = edge_index[1]
    srcm128 = src.reshape(E // KC128, KC128)
    dstm128 = dst.reshape(E // KC128, KC128)
    srcm16 = src.reshape(E // KC16, KC16)
    dstm16 = dst.reshape(E // KC16, KC16)

    hs, hd = _degrees(src, dst)
    dof2, dif2 = _degfin(hs, hd)
    dof = dof2.reshape(N, 1)
    dif = dif2.reshape(N, 1)

    c0, s0, c1, s1 = _cov(x0, x1)
    g0, g1, gb = _fold(c0, s0, c1, s1, params)
    z = _z_kernel(x0, x1, g0, g1, gb, dof)

    aggp = _agg128(z, srcm128, dstm128)
    g = _mid(aggp, dif, dof, params['gcn_b0'][None, :], params['gcn_W1'])
    aggp2 = _agg16(g, srcm16, dstm16)
    return _final(aggp2, dif, params['gcn_b1'][None, :])


# speedup vs baseline: 1.1546x; 1.0000x over previous
import functools

import jax
import jax.numpy as jnp
from jax import lax
from jax.experimental import pallas as pl
from jax.experimental.pallas import tpu as pltpu
from jax.experimental.pallas import tpu_sc as plsc

N = 10000
E = 320000
D_IN = 256
LAT = 64
DEC = 128
HID = 128
NCLS = 16

NC = 2
NS = 16
NW = NC * NS
EPW = E // NW
NPAD = 10240
SLAB = NPAD // NS
WB = 32

_HIGH = jax.lax.Precision.HIGHEST


def _cov_body(x0_ref, x1_ref, c0_ref, s0_ref, c1_ref, s1_ref):
    @pl.when(pl.program_id(0) == 0)
    def _init():
        c0_ref[...] = jnp.zeros_like(c0_ref)
        s0_ref[...] = jnp.zeros_like(s0_ref)
        c1_ref[...] = jnp.zeros_like(c1_ref)
        s1_ref[...] = jnp.zeros_like(s1_ref)

    for x_ref, c_ref, s_ref in ((x0_ref, c0_ref, s0_ref),
                                (x1_ref, c1_ref, s1_ref)):
        x = x_ref[...]
        c_ref[...] += lax.dot_general(x, x, (((0,), (0,)), ((), ())),
                                      preferred_element_type=jnp.float32)
        s_ref[...] += jnp.sum(x, axis=0, keepdims=True)


def _cov(x0, x1):
    blk = 1000
    grid = N // blk
    return pl.pallas_call(
        _cov_body,
        grid=(grid,),
        in_specs=[pl.BlockSpec((blk, D_IN), lambda i: (i, 0)),
                  pl.BlockSpec((blk, D_IN), lambda i: (i, 0))],
        out_specs=[pl.BlockSpec((D_IN, D_IN), lambda i: (0, 0)),
                   pl.BlockSpec((1, D_IN), lambda i: (0, 0)),
                   pl.BlockSpec((D_IN, D_IN), lambda i: (0, 0)),
                   pl.BlockSpec((1, D_IN), lambda i: (0, 0))],
        out_shape=[jax.ShapeDtypeStruct((D_IN, D_IN), jnp.float32),
                   jax.ShapeDtypeStruct((1, D_IN), jnp.float32),
                   jax.ShapeDtypeStruct((D_IN, D_IN), jnp.float32),
                   jax.ShapeDtypeStruct((1, D_IN), jnp.float32)],
    )(x0, x1)


def _fold_body(c0_ref, s0_ref, c1_ref, s1_ref,
               w1_0, b1_0, g1_0, be1_0, w2_0, b2_0, g2_0, be2_0, dw_0, db_0,
               w1_1, b1_1, g1_1, be1_1, w2_1, b2_1, g2_1, be2_1, dw_1, db_1,
               gw0_ref, g0_ref, g1o_ref, gb_ref):
    def mm(a, b):
        return jnp.dot(a, b, preferred_element_type=jnp.float32,
                       precision=_HIGH)

    gw0 = gw0_ref[...]
    bds = None
    for (c_ref, s_ref, W1, b1, g1, be1, W2, b2, g2, be2, dW, db, g_ref) in (
            (c0_ref, s0_ref, w1_0, b1_0, g1_0, be1_0, w2_0, b2_0, g2_0, be2_0,
             dw_0, db_0, g0_ref),
            (c1_ref, s1_ref, w1_1, b1_1, g1_1, be1_1, w2_1, b2_1, g2_1, be2_1,
             dw_1, db_1, g1o_ref)):
        mu = s_ref[...] / N
        cov = c_ref[...] / N - lax.dot_general(
            mu, mu, (((0,), (0,)), ((), ())),
            preferred_element_type=jnp.float32, precision=_HIGH)
        W1v = W1[...]
        mu1 = mm(mu, W1v) + b1[...]
        var1 = jnp.sum(W1v * mm(cov, W1v), axis=0, keepdims=True)
        a1 = g1[...] * lax.rsqrt(var1 + 1e-5)
        d1 = (b1[...] - mu1) * a1 + be1[...]
        W2p = mm(W1v * a1, W2[...])
        b2p = mm(d1, W2[...]) + b2[...]
        mu2 = mm(mu, W2p) + b2p
        var2 = jnp.sum(W2p * mm(cov, W2p), axis=0, keepdims=True)
        a2 = g2[...] * lax.rsqrt(var2 + 1e-5)
        d2 = (b2p - mu2) * a2 + be2[...]
        Wd = mm(W2p * a2, dW[...])
        bd = mm(d2, dW[...]) + db[...]
        g_ref[...] = mm(Wd, gw0) * 0.5
        bds = bd if bds is None else bds + bd
    gb_ref[...] = mm(bds * 0.5, gw0)


def _fold(c0, s0, c1, s1, p):
    args = [c0, s0, c1, s1]
    for m in range(2):
        args += [p[f'enc{m}_W1'], p[f'enc{m}_b1'][None, :],
                 p[f'enc{m}_g1'][None, :], p[f'enc{m}_be1'][None, :],
                 p[f'enc{m}_W2'], p[f'enc{m}_b2'][None, :],
                 p[f'enc{m}_g2'][None, :], p[f'enc{m}_be2'][None, :],
                 p[f'dec{m}_W'], p[f'dec{m}_b'][None, :]]
    args.append(p['gcn_W0'])
    return pl.pallas_call(
        _fold_body,
        out_shape=[jax.ShapeDtypeStruct((D_IN, HID), jnp.float32),
                   jax.ShapeDtypeStruct((D_IN, HID), jnp.float32),
                   jax.ShapeDtypeStruct((1, HID), jnp.float32)],
    )(*args)


def _deg_body(src_ref, dst_ref, hs_ref, hd_ref, idx_v, hist_v):
    cid = lax.axis_index("c")
    sid = lax.axis_index("s")
    wid = cid * NS + sid
    ones = jnp.full((16,), 1.0, jnp.float32)
    zeros = jnp.zeros((16,), jnp.float32)
    for e_ref, h_ref in ((src_ref, hs_ref), (dst_ref, hd_ref)):
        def zero_step(i, _):
            hist_v[pl.ds(i * 16, 16)] = zeros
            return _
        lax.fori_loop(0, N // 16, zero_step, None)
        pltpu.sync_copy(e_ref.at[pl.ds(wid * EPW, EPW)], idx_v)

        def add_step(i, _):
            idx = idx_v[pl.ds(i * 16, 16)]
            plsc.addupdate_scatter(hist_v, [idx], ones)
            return _
        lax.fori_loop(0, EPW // 16, add_step, None)
        pltpu.sync_copy(hist_v, h_ref.at[wid])


def _degrees(src, dst):
    k = pl.kernel(
        _deg_body,
        out_type=[jax.ShapeDtypeStruct((NW, N), jnp.float32),
                  jax.ShapeDtypeStruct((NW, N), jnp.float32)],
        mesh=plsc.VectorSubcoreMesh(core_axis_name="c", subcore_axis_name="s"),
        scratch_types=[pltpu.VMEM((EPW,), jnp.int32),
                       pltpu.VMEM((N,), jnp.float32)],
        compiler_params=pltpu.CompilerParams(needs_layout_passes=False),
    )
    return k(src, dst)


def _degfin_body(hs_ref, hd_ref, do_ref, di_ref):
    s = jnp.sum(hs_ref[...], axis=0, keepdims=True)
    do_ref[...] = lax.rsqrt(jnp.maximum(s, 1.0)).T
    d = jnp.sum(hd_ref[...], axis=0, keepdims=True)
    di_ref[...] = lax.rsqrt(jnp.maximum(d, 1.0)).T


def _degfin(hs, hd):
    return pl.pallas_call(
        _degfin_body,
        out_shape=[jax.ShapeDtypeStruct((N, 1), jnp.float32),
                   jax.ShapeDtypeStruct((N, 1), jnp.float32)],
    )(hs, hd)


def _z_body(x0_ref, x1_ref, g0_ref, g1_ref, gb_ref, do_ref, z_ref):
    z = (jnp.dot(x0_ref[...], g0_ref[...], preferred_element_type=jnp.float32)
         + jnp.dot(x1_ref[...], g1_ref[...],
                   preferred_element_type=jnp.float32)
         + gb_ref[...])
    z_ref[...] = z * do_ref[...]


def _z_kernel(x0, x1, g0, g1, gb, dof):
    blk = 1000
    return pl.pallas_call(
        _z_body,
        grid=(N // blk,),
        in_specs=[pl.BlockSpec((blk, D_IN), lambda i: (i, 0)),
                  pl.BlockSpec((blk, D_IN), lambda i: (i, 0)),
                  pl.BlockSpec((D_IN, HID), lambda i: (0, 0)),
                  pl.BlockSpec((D_IN, HID), lambda i: (0, 0)),
                  pl.BlockSpec((1, HID), lambda i: (0, 0)),
                  pl.BlockSpec((blk, 1), lambda i: (i, 0))],
        out_specs=pl.BlockSpec((blk, HID), lambda i: (i, 0)),
        out_shape=jax.ShapeDtypeStruct((N, HID), jnp.float32),
    )(x0, x1, g0, g1, gb, dof)


def _make_agg(d, kc, nstg):
    nchunk = EPW // kc
    npair = nstg // 2

    def body(tab_ref, srcm_ref, dstm_ref, out_ref, sidx_v, didx_v, rows_a,
             rows_b, wbuf_v, acc_sh, sem_a, sem_b):
        cid = lax.axis_index("c")
        sid = lax.axis_index("s")
        wid = cid * NS + sid
        zeros = jnp.zeros((16,), jnp.float32)

        def zrow(i, _):
            for j in range(d // 16):
                wbuf_v[i, pl.ds(j * 16, 16)] = zeros
            return _
        lax.fori_loop(0, WB, zrow, None)
        for k in range(SLAB // WB):
            pltpu.sync_copy(wbuf_v, acc_sh.at[pl.ds(sid * SLAB + k * WB, WB)])
        plsc.subcore_barrier()

        def sidx(c):
            return sidx_v.at[c]

        def pair(j, _):
            for rows_v, sem, par in ((rows_a, sem_a, 0), (rows_b, sem_b, 1)):
                c = 2 * j + par
                pltpu.make_async_copy(tab_ref.at[sidx(c)], rows_v,
                                      sem).wait()
                pltpu.sync_copy(rows_v, acc_sh.at[didx_v.at[c]], add=True)

                @pl.when(j + 1 < npair)
                def _refill():
                    pltpu.async_copy(tab_ref.at[sidx(c + 2)], rows_v, sem)
            return _

        for s in range(nchunk // nstg):
            base = wid * nchunk + s * nstg
            pltpu.sync_copy(srcm_ref.at[pl.ds(base, nstg)], sidx_v)
            pltpu.sync_copy(dstm_ref.at[pl.ds(base, nstg)], didx_v)
            pltpu.async_copy(tab_ref.at[sidx(0)], rows_a, sem_a)
            pltpu.async_copy(tab_ref.at[sidx(1)], rows_b, sem_b)
            lax.fori_loop(0, npair, pair, None)
        plsc.subcore_barrier()

        for k in range(SLAB // WB):
            off = sid * SLAB + k * WB
            pltpu.sync_copy(acc_sh.at[pl.ds(off, WB)], wbuf_v)
            pltpu.sync_copy(wbuf_v, out_ref.at[cid, pl.ds(off, WB)])

    def agg(table, srcm, dstm):
        k = pl.kernel(
            body,
            out_type=jax.ShapeDtypeStruct((NC, NPAD, d), jnp.float32),
            mesh=plsc.VectorSubcoreMesh(core_axis_name="c",
                                        subcore_axis_name="s"),
            scratch_types=[pltpu.VMEM((nstg, kc), jnp.int32),
                           pltpu.VMEM((nstg, kc), jnp.int32),
                           pltpu.VMEM((kc, d), jnp.float32),
                           pltpu.VMEM((kc, d), jnp.float32),
                           pltpu.VMEM((WB, d), jnp.float32),
                           pltpu.VMEM_SHARED((NPAD, d), jnp.float32),
                           pltpu.SemaphoreType.DMA,
                           pltpu.SemaphoreType.DMA],
            compiler_params=pltpu.CompilerParams(
                needs_layout_passes=False,
                use_tc_tiling_on_sc=(d % 128 == 0)),
        )
        return k(table, srcm, dstm)

    return agg


def _make_agg_ring(d, kc, nstg):
    nchunk = EPW // kc

    def body(tab_ref, srcm_ref, dstm_ref, out_ref, sidx_v, didx_v, b0, b1,
             b2, b3, wbuf_v, acc_sh, sg0, sg1, sg2, sg3, ss0, ss1, ss2, ss3):
        cid = lax.axis_index("c")
        sid = lax.axis_index("s")
        wid = cid * NS + sid
        zeros = jnp.zeros((16,), jnp.float32)
        bufs = (b0, b1, b2, b3)
        sgs = (sg0, sg1, sg2, sg3)
        sss = (ss0, ss1, ss2, ss3)

        def zrow(i, _):
            for j in range(d // 16):
                wbuf_v[i, pl.ds(j * 16, 16)] = zeros
            return _
        lax.fori_loop(0, WB, zrow, None)
        for k in range(SLAB // WB):
            pltpu.sync_copy(wbuf_v, acc_sh.at[pl.ds(sid * SLAB + k * WB, WB)])
        plsc.subcore_barrier()

        def sidx(c):
            return sidx_v.at[c]

        def quad(j, _):
            for q in range(4):
                c = 4 * j + q
                r = (q + 2) % 4
                pltpu.make_async_copy(tab_ref.at[sidx(c)], bufs[q],
                                      sgs[q]).wait()
                pltpu.async_copy(bufs[q], acc_sh.at[didx_v.at[c]], sss[q],
                                 add=True)

                @pl.when((c >= 2) & (c + 2 < nstg))
                def _wait_sc():
                    pltpu.make_async_copy(bufs[r],
                                          acc_sh.at[didx_v.at[c - 2]],
                                          sss[r]).wait()

                @pl.when(c + 2 < nstg)
                def _refill():
                    pltpu.async_copy(tab_ref.at[sidx(c + 2)], bufs[r],
                                     sgs[r])
            return _

        for s in range(nchunk // nstg):
            base = wid * nchunk + s * nstg
            pltpu.sync_copy(srcm_ref.at[pl.ds(base, nstg)], sidx_v)
            pltpu.sync_copy(dstm_ref.at[pl.ds(base, nstg)], didx_v)
            pltpu.async_copy(tab_ref.at[sidx(0)], b0, sg0)
            pltpu.async_copy(tab_ref.at[sidx(1)], b1, sg1)
            lax.fori_loop(0, nstg // 4, quad, None)
            for q in range(4):
                pltpu.make_async_copy(bufs[q],
                                      acc_sh.at[didx_v.at[nstg - 4 + q]],
                                      sss[q]).wait()
        plsc.subcore_barrier()

        for k in range(SLAB // WB):
            off = sid * SLAB + k * WB
            pltpu.sync_copy(acc_sh.at[pl.ds(off, WB)], wbuf_v)
            pltpu.sync_copy(wbuf_v, out_ref.at[cid, pl.ds(off, WB)])

    def agg(table, srcm, dstm):
        k = pl.kernel(
            body,
            out_type=jax.ShapeDtypeStruct((NC, NPAD, d), jnp.float32),
            mesh=plsc.VectorSubcoreMesh(core_axis_name="c",
                                        subcore_axis_name="s"),
            scratch_types=[pltpu.VMEM((nstg, kc), jnp.int32),
                           pltpu.VMEM((nstg, kc), jnp.int32),
                           pltpu.VMEM((kc, d), jnp.float32),
                           pltpu.VMEM((kc, d), jnp.float32),
                           pltpu.VMEM((kc, d), jnp.float32),
                           pltpu.VMEM((kc, d), jnp.float32),
                           pltpu.VMEM((WB, d), jnp.float32),
                           pltpu.VMEM_SHARED((NPAD, d), jnp.float32)]
            + [pltpu.SemaphoreType.DMA] * 8,
            compiler_params=pltpu.CompilerParams(
                needs_layout_passes=False,
                use_tc_tiling_on_sc=(d % 128 == 0)),
        )
        return k(table, srcm, dstm)

    return agg


KC128 = 50
KC16 = 125
_agg128 = _make_agg(HID, KC128, 40)
_agg16 = _make_agg_ring(NCLS, KC16, 16)


def _mid_body(ap_ref, di_ref, do_ref, b0_ref, w1_ref, g_ref):
    s = ap_ref[0] + ap_ref[1]
    h0 = jnp.maximum(s * di_ref[...] + b0_ref[...], 0.0)
    g = jnp.dot(h0, w1_ref[...], preferred_element_type=jnp.float32,
                precision=_HIGH)
    g_ref[...] = g * do_ref[...]


def _mid(aggp, dif, dof, b0, w1):
    blk = 1000
    return pl.pallas_call(
        _mid_body,
        grid=(N // blk,),
        in_specs=[pl.BlockSpec((NC, blk, HID), lambda i: (0, i, 0)),
                  pl.BlockSpec((blk, 1), lambda i: (i, 0)),
                  pl.BlockSpec((blk, 1), lambda i: (i, 0)),
                  pl.BlockSpec((1, HID), lambda i: (0, 0)),
                  pl.BlockSpec((HID, NCLS), lambda i: (0, 0))],
        out_specs=pl.BlockSpec((blk, NCLS), lambda i: (i, 0)),
        out_shape=jax.ShapeDtypeStruct((N, NCLS), jnp.float32),
    )(aggp, dif, dof, b0, w1)


def _fin_body(ap_ref, di_ref, b1_ref, o_ref):
    o_ref[...] = (ap_ref[0] + ap_ref[1]) * di_ref[...] + b1_ref[...]


def _final(aggp2, dif, b1):
    blk = 1000
    return pl.pallas_call(
        _fin_body,
        grid=(N // blk,),
        in_specs=[pl.BlockSpec((NC, blk, NCLS), lambda i: (0, i, 0)),
                  pl.BlockSpec((blk, 1), lambda i: (i, 0)),
                  pl.BlockSpec((1, NCLS), lambda i: (0, 0))],
        out_specs=pl.BlockSpec((blk, NCLS), lambda i: (i, 0)),
        out_shape=jax.ShapeDtypeStruct((N, NCLS), jnp.float32),
    )(aggp2, dif, b1)


def kernel(x0, x1, params, edge_index):
    src = edge_index[0]
    dst = edge_index[1]
    srcm128 = src.reshape(E // KC128, KC128)
    dstm128 = dst.reshape(E // KC128, KC128)
    srcm16 = src.reshape(E // KC16, KC16)
    dstm16 = dst.reshape(E // KC16, KC16)

    hs, hd = _degrees(src, dst)
    dof, dif = _degfin(hs, hd)

    c0, s0, c1, s1 = _cov(x0, x1)
    g0, g1, gb = _fold(c0, s0, c1, s1, params)
    z = _z_kernel(x0, x1, g0, g1, gb, dof)

    aggp = _agg128(z, srcm128, dstm128)
    g = _mid(aggp, dif, dof, params['gcn_b0'][None, :], params['gcn_W1'])
    aggp2 = _agg16(g, srcm16, dstm16)
    return _final(aggp2, dif, params['gcn_b1'][None, :])


# speedup vs baseline: 1.1811x; 1.0230x over previous
import functools

import jax
import jax.numpy as jnp
from jax import lax
from jax.experimental import pallas as pl
from jax.experimental.pallas import tpu as pltpu
from jax.experimental.pallas import tpu_sc as plsc

N = 10000
E = 320000
D_IN = 256
LAT = 64
DEC = 128
HID = 128
NCLS = 16

NC = 2
NS = 16
NW = NC * NS
EPW = E // NW
NPAD = 10240
SLAB = NPAD // NS
WB = 32

_HIGH = jax.lax.Precision.HIGHEST


def _cov_body(x0_ref, x1_ref, c0_ref, s0_ref, c1_ref, s1_ref):
    @pl.when(pl.program_id(0) == 0)
    def _init():
        c0_ref[...] = jnp.zeros_like(c0_ref)
        s0_ref[...] = jnp.zeros_like(s0_ref)
        c1_ref[...] = jnp.zeros_like(c1_ref)
        s1_ref[...] = jnp.zeros_like(s1_ref)

    for x_ref, c_ref, s_ref in ((x0_ref, c0_ref, s0_ref),
                                (x1_ref, c1_ref, s1_ref)):
        x = x_ref[...]
        c_ref[...] += lax.dot_general(x, x, (((0,), (0,)), ((), ())),
                                      preferred_element_type=jnp.float32)
        s_ref[...] += jnp.sum(x, axis=0, keepdims=True)


def _cov(x0, x1):
    blk = 1000
    grid = N // blk
    return pl.pallas_call(
        _cov_body,
        grid=(grid,),
        in_specs=[pl.BlockSpec((blk, D_IN), lambda i: (i, 0)),
                  pl.BlockSpec((blk, D_IN), lambda i: (i, 0))],
        out_specs=[pl.BlockSpec((D_IN, D_IN), lambda i: (0, 0)),
                   pl.BlockSpec((1, D_IN), lambda i: (0, 0)),
                   pl.BlockSpec((D_IN, D_IN), lambda i: (0, 0)),
                   pl.BlockSpec((1, D_IN), lambda i: (0, 0))],
        out_shape=[jax.ShapeDtypeStruct((D_IN, D_IN), jnp.float32),
                   jax.ShapeDtypeStruct((1, D_IN), jnp.float32),
                   jax.ShapeDtypeStruct((D_IN, D_IN), jnp.float32),
                   jax.ShapeDtypeStruct((1, D_IN), jnp.float32)],
    )(x0, x1)


def _fold_body(c0_ref, s0_ref, c1_ref, s1_ref,
               w1_0, b1_0, g1_0, be1_0, w2_0, b2_0, g2_0, be2_0, dw_0, db_0,
               w1_1, b1_1, g1_1, be1_1, w2_1, b2_1, g2_1, be2_1, dw_1, db_1,
               gw0_ref, g0_ref, g1o_ref, gb_ref):
    def mm(a, b):
        return jnp.dot(a, b, preferred_element_type=jnp.float32,
                       precision=_HIGH)

    gw0 = gw0_ref[...]
    bds = None
    for (c_ref, s_ref, W1, b1, g1, be1, W2, b2, g2, be2, dW, db, g_ref) in (
            (c0_ref, s0_ref, w1_0, b1_0, g1_0, be1_0, w2_0, b2_0, g2_0, be2_0,
             dw_0, db_0, g0_ref),
            (c1_ref, s1_ref, w1_1, b1_1, g1_1, be1_1, w2_1, b2_1, g2_1, be2_1,
             dw_1, db_1, g1o_ref)):
        mu = s_ref[...] / N
        cov = c_ref[...] / N - lax.dot_general(
            mu, mu, (((0,), (0,)), ((), ())),
            preferred_element_type=jnp.float32, precision=_HIGH)
        W1v = W1[...]
        mu1 = mm(mu, W1v) + b1[...]
        var1 = jnp.sum(W1v * mm(cov, W1v), axis=0, keepdims=True)
        a1 = g1[...] * lax.rsqrt(var1 + 1e-5)
        d1 = (b1[...] - mu1) * a1 + be1[...]
        W2p = mm(W1v * a1, W2[...])
        b2p = mm(d1, W2[...]) + b2[...]
        mu2 = mm(mu, W2p) + b2p
        var2 = jnp.sum(W2p * mm(cov, W2p), axis=0, keepdims=True)
        a2 = g2[...] * lax.rsqrt(var2 + 1e-5)
        d2 = (b2p - mu2) * a2 + be2[...]
        Wd = mm(W2p * a2, dW[...])
        bd = mm(d2, dW[...]) + db[...]
        g_ref[...] = mm(Wd, gw0) * 0.5
        bds = bd if bds is None else bds + bd
    gb_ref[...] = mm(bds * 0.5, gw0)


def _fold(c0, s0, c1, s1, p):
    args = [c0, s0, c1, s1]
    for m in range(2):
        args += [p[f'enc{m}_W1'], p[f'enc{m}_b1'][None, :],
                 p[f'enc{m}_g1'][None, :], p[f'enc{m}_be1'][None, :],
                 p[f'enc{m}_W2'], p[f'enc{m}_b2'][None, :],
                 p[f'enc{m}_g2'][None, :], p[f'enc{m}_be2'][None, :],
                 p[f'dec{m}_W'], p[f'dec{m}_b'][None, :]]
    args.append(p['gcn_W0'])
    return pl.pallas_call(
        _fold_body,
        out_shape=[jax.ShapeDtypeStruct((D_IN, HID), jnp.float32),
                   jax.ShapeDtypeStruct((D_IN, HID), jnp.float32),
                   jax.ShapeDtypeStruct((1, HID), jnp.float32)],
    )(*args)


def _deg_body(src_ref, dst_ref, hs_ref, hd_ref, idx_v, hist_v):
    cid = lax.axis_index("c")
    sid = lax.axis_index("s")
    wid = cid * NS + sid
    ones = jnp.full((16,), 1.0, jnp.float32)
    zeros = jnp.zeros((16,), jnp.float32)
    for e_ref, h_ref in ((src_ref, hs_ref), (dst_ref, hd_ref)):
        def zero_step(i, _):
            hist_v[pl.ds(i * 16, 16)] = zeros
            return _
        lax.fori_loop(0, N // 16, zero_step, None)
        pltpu.sync_copy(e_ref.at[pl.ds(wid * EPW, EPW)], idx_v)

        def add_step(i, _):
            idx = idx_v[pl.ds(i * 16, 16)]
            plsc.addupdate_scatter(hist_v, [idx], ones)
            return _
        lax.fori_loop(0, EPW // 16, add_step, None)
        pltpu.sync_copy(hist_v, h_ref.at[wid])


def _degrees(src, dst):
    k = pl.kernel(
        _deg_body,
        out_type=[jax.ShapeDtypeStruct((NW, N), jnp.float32),
                  jax.ShapeDtypeStruct((NW, N), jnp.float32)],
        mesh=plsc.VectorSubcoreMesh(core_axis_name="c", subcore_axis_name="s"),
        scratch_types=[pltpu.VMEM((EPW,), jnp.int32),
                       pltpu.VMEM((N,), jnp.float32)],
        compiler_params=pltpu.CompilerParams(needs_layout_passes=False),
    )
    return k(src, dst)


def _degfin_body(hs_ref, hd_ref, do_ref, di_ref):
    s = jnp.sum(hs_ref[...], axis=0, keepdims=True)
    do_ref[...] = lax.rsqrt(jnp.maximum(s, 1.0)).T
    d = jnp.sum(hd_ref[...], axis=0, keepdims=True)
    di_ref[...] = lax.rsqrt(jnp.maximum(d, 1.0)).T


def _degfin(hs, hd):
    return pl.pallas_call(
        _degfin_body,
        out_shape=[jax.ShapeDtypeStruct((N, 1), jnp.float32),
                   jax.ShapeDtypeStruct((N, 1), jnp.float32)],
    )(hs, hd)


def _z_body(x0_ref, x1_ref, g0_ref, g1_ref, gb_ref, do_ref, z_ref):
    z = (jnp.dot(x0_ref[...], g0_ref[...], preferred_element_type=jnp.float32)
         + jnp.dot(x1_ref[...], g1_ref[...],
                   preferred_element_type=jnp.float32)
         + gb_ref[...])
    z_ref[...] = z * do_ref[...]


def _z_kernel(x0, x1, g0, g1, gb, dof):
    blk = 1000
    return pl.pallas_call(
        _z_body,
        grid=(N // blk,),
        in_specs=[pl.BlockSpec((blk, D_IN), lambda i: (i, 0)),
                  pl.BlockSpec((blk, D_IN), lambda i: (i, 0)),
                  pl.BlockSpec((D_IN, HID), lambda i: (0, 0)),
                  pl.BlockSpec((D_IN, HID), lambda i: (0, 0)),
                  pl.BlockSpec((1, HID), lambda i: (0, 0)),
                  pl.BlockSpec((blk, 1), lambda i: (i, 0))],
        out_specs=pl.BlockSpec((blk, HID), lambda i: (i, 0)),
        out_shape=jax.ShapeDtypeStruct((N, HID), jnp.float32),
    )(x0, x1, g0, g1, gb, dof)


def _make_agg(d, kc, nstg):
    nchunk = EPW // kc
    npair = nstg // 2

    def body(tab_ref, srcm_ref, dstm_ref, out_ref, sidx_v, didx_v, rows_a,
             rows_b, wbuf_v, acc_sh, sem_a, sem_b):
        cid = lax.axis_index("c")
        sid = lax.axis_index("s")
        wid = cid * NS + sid
        zeros = jnp.zeros((16,), jnp.float32)

        def zrow(i, _):
            for j in range(d // 16):
                wbuf_v[i, pl.ds(j * 16, 16)] = zeros
            return _
        lax.fori_loop(0, WB, zrow, None)
        for k in range(SLAB // WB):
            pltpu.sync_copy(wbuf_v, acc_sh.at[pl.ds(sid * SLAB + k * WB, WB)])
        plsc.subcore_barrier()

        def sidx(c):
            return sidx_v.at[c]

        def pair(j, _):
            for rows_v, sem, par in ((rows_a, sem_a, 0), (rows_b, sem_b, 1)):
                c = 2 * j + par
                pltpu.make_async_copy(tab_ref.at[sidx(c)], rows_v,
                                      sem).wait()
                pltpu.sync_copy(rows_v, acc_sh.at[didx_v.at[c]], add=True)

                @pl.when(j + 1 < npair)
                def _refill():
                    pltpu.async_copy(tab_ref.at[sidx(c + 2)], rows_v, sem)
            return _

        for s in range(nchunk // nstg):
            base = wid * nchunk + s * nstg
            pltpu.sync_copy(srcm_ref.at[pl.ds(base, nstg)], sidx_v)
            pltpu.sync_copy(dstm_ref.at[pl.ds(base, nstg)], didx_v)
            pltpu.async_copy(tab_ref.at[sidx(0)], rows_a, sem_a)
            pltpu.async_copy(tab_ref.at[sidx(1)], rows_b, sem_b)
            lax.fori_loop(0, npair, pair, None)
        plsc.subcore_barrier()

        for k in range(SLAB // WB):
            off = sid * SLAB + k * WB
            pltpu.sync_copy(acc_sh.at[pl.ds(off, WB)], wbuf_v)
            pltpu.sync_copy(wbuf_v, out_ref.at[cid, pl.ds(off, WB)])

    def agg(table, srcm, dstm):
        k = pl.kernel(
            body,
            out_type=jax.ShapeDtypeStruct((NC, NPAD, d), jnp.float32),
            mesh=plsc.VectorSubcoreMesh(core_axis_name="c",
                                        subcore_axis_name="s"),
            scratch_types=[pltpu.VMEM((nstg, kc), jnp.int32),
                           pltpu.VMEM((nstg, kc), jnp.int32),
                           pltpu.VMEM((kc, d), jnp.float32),
                           pltpu.VMEM((kc, d), jnp.float32),
                           pltpu.VMEM((WB, d), jnp.float32),
                           pltpu.VMEM_SHARED((NPAD, d), jnp.float32),
                           pltpu.SemaphoreType.DMA,
                           pltpu.SemaphoreType.DMA],
            compiler_params=pltpu.CompilerParams(
                needs_layout_passes=False,
                use_tc_tiling_on_sc=(d % 128 == 0)),
        )
        return k(table, srcm, dstm)

    return agg


def _make_agg_ring(d, kc, nstg):
    nchunk = EPW // kc

    def body(tab_ref, srcm_ref, dstm_ref, out_ref, sidx_v, didx_v, b0, b1,
             b2, b3, wbuf_v, acc_sh, sg0, sg1, sg2, sg3, ss0, ss1, ss2, ss3):
        cid = lax.axis_index("c")
        sid = lax.axis_index("s")
        wid = cid * NS + sid
        zeros = jnp.zeros((16,), jnp.float32)
        bufs = (b0, b1, b2, b3)
        sgs = (sg0, sg1, sg2, sg3)
        sss = (ss0, ss1, ss2, ss3)

        def zrow(i, _):
            for j in range(d // 16):
                wbuf_v[i, pl.ds(j * 16, 16)] = zeros
            return _
        lax.fori_loop(0, WB, zrow, None)
        for k in range(SLAB // WB):
            pltpu.sync_copy(wbuf_v, acc_sh.at[pl.ds(sid * SLAB + k * WB, WB)])
        plsc.subcore_barrier()

        def sidx(c):
            return sidx_v.at[c]

        def quad(j, _):
            for q in range(4):
                c = 4 * j + q
                r = (q + 2) % 4
                pltpu.make_async_copy(tab_ref.at[sidx(c)], bufs[q],
                                      sgs[q]).wait()
                pltpu.async_copy(bufs[q], acc_sh.at[didx_v.at[c]], sss[q],
                                 add=True)

                @pl.when((c >= 2) & (c + 2 < nstg))
                def _wait_sc():
                    pltpu.make_async_copy(bufs[r],
                                          acc_sh.at[didx_v.at[c - 2]],
                                          sss[r]).wait()

                @pl.when(c + 2 < nstg)
                def _refill():
                    pltpu.async_copy(tab_ref.at[sidx(c + 2)], bufs[r],
                                     sgs[r])
            return _

        for s in range(nchunk // nstg):
            base = wid * nchunk + s * nstg
            pltpu.sync_copy(srcm_ref.at[pl.ds(base, nstg)], sidx_v)
            pltpu.sync_copy(dstm_ref.at[pl.ds(base, nstg)], didx_v)
            pltpu.async_copy(tab_ref.at[sidx(0)], b0, sg0)
            pltpu.async_copy(tab_ref.at[sidx(1)], b1, sg1)
            lax.fori_loop(0, nstg // 4, quad, None)
            for q in range(4):
                pltpu.make_async_copy(bufs[q],
                                      acc_sh.at[didx_v.at[nstg - 4 + q]],
                                      sss[q]).wait()
        plsc.subcore_barrier()

        for k in range(SLAB // WB):
            off = sid * SLAB + k * WB
            pltpu.sync_copy(acc_sh.at[pl.ds(off, WB)], wbuf_v)
            pltpu.sync_copy(wbuf_v, out_ref.at[cid, pl.ds(off, WB)])

    def agg(table, srcm, dstm):
        k = pl.kernel(
            body,
            out_type=jax.ShapeDtypeStruct((NC, NPAD, d), jnp.float32),
            mesh=plsc.VectorSubcoreMesh(core_axis_name="c",
                                        subcore_axis_name="s"),
            scratch_types=[pltpu.VMEM((nstg, kc), jnp.int32),
                           pltpu.VMEM((nstg, kc), jnp.int32),
                           pltpu.VMEM((kc, d), jnp.float32),
                           pltpu.VMEM((kc, d), jnp.float32),
                           pltpu.VMEM((kc, d), jnp.float32),
                           pltpu.VMEM((kc, d), jnp.float32),
                           pltpu.VMEM((WB, d), jnp.float32),
                           pltpu.VMEM_SHARED((NPAD, d), jnp.float32)]
            + [pltpu.SemaphoreType.DMA] * 8,
            compiler_params=pltpu.CompilerParams(
                needs_layout_passes=False,
                use_tc_tiling_on_sc=(d % 128 == 0)),
        )
        return k(table, srcm, dstm)

    return agg


KC128 = 50
KC16 = 125
_agg128 = _make_agg(HID, KC128, 40)
_agg16 = _make_agg_ring(NCLS, KC16, 16)


def _mid_body(ap_ref, di_ref, do_ref, b0_ref, w1_ref, g_ref):
    s = ap_ref[0] + ap_ref[1]
    h0 = jnp.maximum(s * di_ref[...] + b0_ref[...], 0.0)
    g = jnp.dot(h0, w1_ref[...], preferred_element_type=jnp.float32,
                precision=_HIGH)
    g_ref[...] = g * do_ref[...]


def _mid(aggp, dif, dof, b0, w1):
    blk = 1000
    return pl.pallas_call(
        _mid_body,
        grid=(N // blk,),
        in_specs=[pl.BlockSpec((NC, blk, HID), lambda i: (0, i, 0)),
                  pl.BlockSpec((blk, 1), lambda i: (i, 0)),
                  pl.BlockSpec((blk, 1), lambda i: (i, 0)),
                  pl.BlockSpec((1, HID), lambda i: (0, 0)),
                  pl.BlockSpec((HID, NCLS), lambda i: (0, 0))],
        out_specs=pl.BlockSpec((blk, NCLS), lambda i: (i, 0)),
        out_shape=jax.ShapeDtypeStruct((N, NCLS), jnp.float32),
    )(aggp, dif, dof, b0, w1)


PACK = 128 // NCLS
NROWS = NPAD // PACK


def _fin_body(ap_ref, dp_ref, b1_ref, o_ref):
    r = lax.broadcasted_iota(jnp.int32, (PACK, 128), 0)
    l = lax.broadcasted_iota(jnp.int32, (PACK, 128), 1)
    hot = jnp.where(l // NCLS == r, 1.0, 0.0).astype(jnp.float32)
    dp = jnp.dot(dp_ref[...], hot, preferred_element_type=jnp.float32)
    o_ref[...] = (ap_ref[0] + ap_ref[1]) * dp + b1_ref[...]


def _final(aggp2r, difp, b1p):
    blk = 128
    return pl.pallas_call(
        _fin_body,
        grid=(NROWS // blk,),
        in_specs=[pl.BlockSpec((NC, blk, 128), lambda i: (0, i, 0)),
                  pl.BlockSpec((blk, PACK), lambda i: (i, 0)),
                  pl.BlockSpec((1, 128), lambda i: (0, 0))],
        out_specs=pl.BlockSpec((blk, 128), lambda i: (i, 0)),
        out_shape=jax.ShapeDtypeStruct((NROWS, 128), jnp.float32),
    )(aggp2r, difp, b1p)


def kernel(x0, x1, params, edge_index):
    src = edge_index[0]
    dst = edge_index[1]
    srcm128 = src.reshape(E // KC128, KC128)
    dstm128 = dst.reshape(E // KC128, KC128)
    srcm16 = src.reshape(E // KC16, KC16)
    dstm16 = dst.reshape(E // KC16, KC16)

    hs, hd = _degrees(src, dst)
    dof, dif = _degfin(hs, hd)

    c0, s0, c1, s1 = _cov(x0, x1)
    g0, g1, gb = _fold(c0, s0, c1, s1, params)
    z = _z_kernel(x0, x1, g0, g1, gb, dof)

    aggp = _agg128(z, srcm128, dstm128)
    g = _mid(aggp, dif, dof, params['gcn_b0'][None, :], params['gcn_W1'])
    aggp2 = _agg16(g, srcm16, dstm16)
    aggp2r = aggp2.reshape(NC, NROWS, 128)
    difp = jnp.pad(dif.reshape(N // PACK, PACK),
                   ((0, NROWS - N // PACK), (0, 0)))
    b1p = jnp.tile(params['gcn_b1'], PACK)[None, :]
    fin = _final(aggp2r, difp, b1p)
    return fin.reshape(NPAD, NCLS)[:N]


# speedup vs baseline: 1.1840x; 1.0024x over previous
import functools

import jax
import jax.numpy as jnp
from jax import lax
from jax.experimental import pallas as pl
from jax.experimental.pallas import tpu as pltpu
from jax.experimental.pallas import tpu_sc as plsc

N = 10000
E = 320000
D_IN = 256
LAT = 64
DEC = 128
HID = 128
NCLS = 16

NC = 2
NS = 16
NW = NC * NS
EPW = E // NW
NPAD = 10240
SLAB = NPAD // NS
WB = 32

_HIGH = jax.lax.Precision.HIGHEST


def _cov_body(x0_ref, x1_ref, c0_ref, s0_ref, c1_ref, s1_ref):
    @pl.when(pl.program_id(0) == 0)
    def _init():
        c0_ref[...] = jnp.zeros_like(c0_ref)
        s0_ref[...] = jnp.zeros_like(s0_ref)
        c1_ref[...] = jnp.zeros_like(c1_ref)
        s1_ref[...] = jnp.zeros_like(s1_ref)

    for x_ref, c_ref, s_ref in ((x0_ref, c0_ref, s0_ref),
                                (x1_ref, c1_ref, s1_ref)):
        x = x_ref[...]
        c_ref[...] += lax.dot_general(x, x, (((0,), (0,)), ((), ())),
                                      preferred_element_type=jnp.float32)
        s_ref[...] += jnp.sum(x, axis=0, keepdims=True)


def _cov(x0, x1):
    blk = 1000
    grid = N // blk
    return pl.pallas_call(
        _cov_body,
        grid=(grid,),
        in_specs=[pl.BlockSpec((blk, D_IN), lambda i: (i, 0)),
                  pl.BlockSpec((blk, D_IN), lambda i: (i, 0))],
        out_specs=[pl.BlockSpec((D_IN, D_IN), lambda i: (0, 0)),
                   pl.BlockSpec((1, D_IN), lambda i: (0, 0)),
                   pl.BlockSpec((D_IN, D_IN), lambda i: (0, 0)),
                   pl.BlockSpec((1, D_IN), lambda i: (0, 0))],
        out_shape=[jax.ShapeDtypeStruct((D_IN, D_IN), jnp.float32),
                   jax.ShapeDtypeStruct((1, D_IN), jnp.float32),
                   jax.ShapeDtypeStruct((D_IN, D_IN), jnp.float32),
                   jax.ShapeDtypeStruct((1, D_IN), jnp.float32)],
    )(x0, x1)


def _fold_body(c0_ref, s0_ref, c1_ref, s1_ref,
               w1_0, b1_0, g1_0, be1_0, w2_0, b2_0, g2_0, be2_0, dw_0, db_0,
               w1_1, b1_1, g1_1, be1_1, w2_1, b2_1, g2_1, be2_1, dw_1, db_1,
               gw0_ref, g0_ref, g1o_ref, gb_ref):
    def mm(a, b):
        return jnp.dot(a, b, preferred_element_type=jnp.float32,
                       precision=_HIGH)

    gw0 = gw0_ref[...]
    bds = None
    for (c_ref, s_ref, W1, b1, g1, be1, W2, b2, g2, be2, dW, db, g_ref) in (
            (c0_ref, s0_ref, w1_0, b1_0, g1_0, be1_0, w2_0, b2_0, g2_0, be2_0,
             dw_0, db_0, g0_ref),
            (c1_ref, s1_ref, w1_1, b1_1, g1_1, be1_1, w2_1, b2_1, g2_1, be2_1,
             dw_1, db_1, g1o_ref)):
        mu = s_ref[...] / N
        cov = c_ref[...] / N - lax.dot_general(
            mu, mu, (((0,), (0,)), ((), ())),
            preferred_element_type=jnp.float32, precision=_HIGH)
        W1v = W1[...]
        mu1 = mm(mu, W1v) + b1[...]
        var1 = jnp.sum(W1v * mm(cov, W1v), axis=0, keepdims=True)
        a1 = g1[...] * lax.rsqrt(var1 + 1e-5)
        d1 = (b1[...] - mu1) * a1 + be1[...]
        W2p = mm(W1v * a1, W2[...])
        b2p = mm(d1, W2[...]) + b2[...]
        mu2 = mm(mu, W2p) + b2p
        var2 = jnp.sum(W2p * mm(cov, W2p), axis=0, keepdims=True)
        a2 = g2[...] * lax.rsqrt(var2 + 1e-5)
        d2 = (b2p - mu2) * a2 + be2[...]
        Wd = mm(W2p * a2, dW[...])
        bd = mm(d2, dW[...]) + db[...]
        g_ref[...] = mm(Wd, gw0) * 0.5
        bds = bd if bds is None else bds + bd
    gb_ref[...] = mm(bds * 0.5, gw0)


def _fold(c0, s0, c1, s1, p):
    args = [c0, s0, c1, s1]
    for m in range(2):
        args += [p[f'enc{m}_W1'], p[f'enc{m}_b1'][None, :],
                 p[f'enc{m}_g1'][None, :], p[f'enc{m}_be1'][None, :],
                 p[f'enc{m}_W2'], p[f'enc{m}_b2'][None, :],
                 p[f'enc{m}_g2'][None, :], p[f'enc{m}_be2'][None, :],
                 p[f'dec{m}_W'], p[f'dec{m}_b'][None, :]]
    args.append(p['gcn_W0'])
    return pl.pallas_call(
        _fold_body,
        out_shape=[jax.ShapeDtypeStruct((D_IN, HID), jnp.float32),
                   jax.ShapeDtypeStruct((D_IN, HID), jnp.float32),
                   jax.ShapeDtypeStruct((1, HID), jnp.float32)],
    )(*args)


def _deg_body(src_ref, dst_ref, hs_ref, hd_ref, idx_v, hist_v):
    cid = lax.axis_index("c")
    sid = lax.axis_index("s")
    wid = cid * NS + sid
    ones = jnp.full((16,), 1.0, jnp.float32)
    zeros = jnp.zeros((16,), jnp.float32)
    for e_ref, h_ref in ((src_ref, hs_ref), (dst_ref, hd_ref)):
        def zero_step(i, _):
            hist_v[pl.ds(i * 16, 16)] = zeros
            return _
        lax.fori_loop(0, N // 16, zero_step, None)
        pltpu.sync_copy(e_ref.at[pl.ds(wid * EPW, EPW)], idx_v)

        def add_step(i, _):
            idx = idx_v[pl.ds(i * 16, 16)]
            plsc.addupdate_scatter(hist_v, [idx], ones)
            return _
        lax.fori_loop(0, EPW // 16, add_step, None)
        pltpu.sync_copy(hist_v, h_ref.at[wid])


def _degrees(src, dst):
    k = pl.kernel(
        _deg_body,
        out_type=[jax.ShapeDtypeStruct((NW, N), jnp.float32),
                  jax.ShapeDtypeStruct((NW, N), jnp.float32)],
        mesh=plsc.VectorSubcoreMesh(core_axis_name="c", subcore_axis_name="s"),
        scratch_types=[pltpu.VMEM((EPW,), jnp.int32),
                       pltpu.VMEM((N,), jnp.float32)],
        compiler_params=pltpu.CompilerParams(needs_layout_passes=False),
    )
    return k(src, dst)


def _degfin_body(hs_ref, hd_ref, do_ref, di_ref):
    pad = jnp.zeros((1, NPAD - N), jnp.float32)
    s = jnp.sum(hs_ref[...], axis=0, keepdims=True)
    f = jnp.concatenate([lax.rsqrt(jnp.maximum(s, 1.0)), pad], axis=1)
    do_ref[...] = f.T
    d = jnp.sum(hd_ref[...], axis=0, keepdims=True)
    g = jnp.concatenate([lax.rsqrt(jnp.maximum(d, 1.0)), pad], axis=1)
    di_ref[...] = g.T


def _degfin(hs, hd):
    return pl.pallas_call(
        _degfin_body,
        out_shape=[jax.ShapeDtypeStruct((NPAD, 1), jnp.float32),
                   jax.ShapeDtypeStruct((NPAD, 1), jnp.float32)],
    )(hs, hd)


def _z_body(x0_ref, x1_ref, g0_ref, g1_ref, gb_ref, do_ref, z_ref):
    z = (jnp.dot(x0_ref[...], g0_ref[...], preferred_element_type=jnp.float32)
         + jnp.dot(x1_ref[...], g1_ref[...],
                   preferred_element_type=jnp.float32)
         + gb_ref[...])
    z_ref[...] = z * do_ref[...]


def _z_kernel(x0, x1, g0, g1, gb, dof):
    blk = 1000
    return pl.pallas_call(
        _z_body,
        grid=(N // blk,),
        in_specs=[pl.BlockSpec((blk, D_IN), lambda i: (i, 0)),
                  pl.BlockSpec((blk, D_IN), lambda i: (i, 0)),
                  pl.BlockSpec((D_IN, HID), lambda i: (0, 0)),
                  pl.BlockSpec((D_IN, HID), lambda i: (0, 0)),
                  pl.BlockSpec((1, HID), lambda i: (0, 0)),
                  pl.BlockSpec((blk, 1), lambda i: (i, 0))],
        out_specs=pl.BlockSpec((blk, HID), lambda i: (i, 0)),
        out_shape=jax.ShapeDtypeStruct((N, HID), jnp.float32),
    )(x0, x1, g0, g1, gb, dof)


def _make_agg(d, kc, nstg):
    nchunk = EPW // kc
    npair = nstg // 2

    def body(tab_ref, srcm_ref, dstm_ref, out_ref, sidx_v, didx_v, rows_a,
             rows_b, wbuf_v, acc_sh, sem_a, sem_b):
        cid = lax.axis_index("c")
        sid = lax.axis_index("s")
        wid = cid * NS + sid
        zeros = jnp.zeros((16,), jnp.float32)

        def zrow(i, _):
            for j in range(d // 16):
                wbuf_v[i, pl.ds(j * 16, 16)] = zeros
            return _
        lax.fori_loop(0, WB, zrow, None)
        for k in range(SLAB // WB):
            pltpu.sync_copy(wbuf_v, acc_sh.at[pl.ds(sid * SLAB + k * WB, WB)])
        plsc.subcore_barrier()

        def sidx(c):
            return sidx_v.at[c]

        def pair(j, _):
            for rows_v, sem, par in ((rows_a, sem_a, 0), (rows_b, sem_b, 1)):
                c = 2 * j + par
                pltpu.make_async_copy(tab_ref.at[sidx(c)], rows_v,
                                      sem).wait()
                pltpu.sync_copy(rows_v, acc_sh.at[didx_v.at[c]], add=True)

                @pl.when(j + 1 < npair)
                def _refill():
                    pltpu.async_copy(tab_ref.at[sidx(c + 2)], rows_v, sem)
            return _

        for s in range(nchunk // nstg):
            base = wid * nchunk + s * nstg
            pltpu.sync_copy(srcm_ref.at[pl.ds(base, nstg)], sidx_v)
            pltpu.sync_copy(dstm_ref.at[pl.ds(base, nstg)], didx_v)
            pltpu.async_copy(tab_ref.at[sidx(0)], rows_a, sem_a)
            pltpu.async_copy(tab_ref.at[sidx(1)], rows_b, sem_b)
            lax.fori_loop(0, npair, pair, None)
        plsc.subcore_barrier()

        for k in range(SLAB // WB):
            off = sid * SLAB + k * WB
            pltpu.sync_copy(acc_sh.at[pl.ds(off, WB)], wbuf_v)
            pltpu.sync_copy(wbuf_v, out_ref.at[cid, pl.ds(off, WB)])

    def agg(table, srcm, dstm):
        k = pl.kernel(
            body,
            out_type=jax.ShapeDtypeStruct((NC, NPAD, d), jnp.float32),
            mesh=plsc.VectorSubcoreMesh(core_axis_name="c",
                                        subcore_axis_name="s"),
            scratch_types=[pltpu.VMEM((nstg, kc), jnp.int32),
                           pltpu.VMEM((nstg, kc), jnp.int32),
                           pltpu.VMEM((kc, d), jnp.float32),
                           pltpu.VMEM((kc, d), jnp.float32),
                           pltpu.VMEM((WB, d), jnp.float32),
                           pltpu.VMEM_SHARED((NPAD, d), jnp.float32),
                           pltpu.SemaphoreType.DMA,
                           pltpu.SemaphoreType.DMA],
            compiler_params=pltpu.CompilerParams(
                needs_layout_passes=False,
                use_tc_tiling_on_sc=(d % 128 == 0)),
        )
        return k(table, srcm, dstm)

    return agg


def _make_agg_ring(d, kc, nstg):
    nchunk = EPW // kc

    def body(tab_ref, srcm_ref, dstm_ref, out_ref, sidx_v, didx_v, b0, b1,
             b2, b3, wbuf_v, acc_sh, sg0, sg1, sg2, sg3, ss0, ss1, ss2, ss3):
        cid = lax.axis_index("c")
        sid = lax.axis_index("s")
        wid = cid * NS + sid
        zeros = jnp.zeros((16,), jnp.float32)
        bufs = (b0, b1, b2, b3)
        sgs = (sg0, sg1, sg2, sg3)
        sss = (ss0, ss1, ss2, ss3)

        def zrow(i, _):
            for j in range(d // 16):
                wbuf_v[i, pl.ds(j * 16, 16)] = zeros
            return _
        lax.fori_loop(0, WB, zrow, None)
        for k in range(SLAB // WB):
            pltpu.sync_copy(wbuf_v, acc_sh.at[pl.ds(sid * SLAB + k * WB, WB)])
        plsc.subcore_barrier()

        def sidx(c):
            return sidx_v.at[c]

        def quad(j, _):
            for q in range(4):
                c = 4 * j + q
                r = (q + 2) % 4
                pltpu.make_async_copy(tab_ref.at[sidx(c)], bufs[q],
                                      sgs[q]).wait()
                pltpu.async_copy(bufs[q], acc_sh.at[didx_v.at[c]], sss[q],
                                 add=True)

                @pl.when((c >= 2) & (c + 2 < nstg))
                def _wait_sc():
                    pltpu.make_async_copy(bufs[r],
                                          acc_sh.at[didx_v.at[c - 2]],
                                          sss[r]).wait()

                @pl.when(c + 2 < nstg)
                def _refill():
                    pltpu.async_copy(tab_ref.at[sidx(c + 2)], bufs[r],
                                     sgs[r])
            return _

        for s in range(nchunk // nstg):
            base = wid * nchunk + s * nstg
            pltpu.sync_copy(srcm_ref.at[pl.ds(base, nstg)], sidx_v)
            pltpu.sync_copy(dstm_ref.at[pl.ds(base, nstg)], didx_v)
            pltpu.async_copy(tab_ref.at[sidx(0)], b0, sg0)
            pltpu.async_copy(tab_ref.at[sidx(1)], b1, sg1)
            lax.fori_loop(0, nstg // 4, quad, None)
            for q in range(4):
                pltpu.make_async_copy(bufs[q],
                                      acc_sh.at[didx_v.at[nstg - 4 + q]],
                                      sss[q]).wait()
        plsc.subcore_barrier()

        for k in range(SLAB // WB):
            off = sid * SLAB + k * WB
            pltpu.sync_copy(acc_sh.at[pl.ds(off, WB)], wbuf_v)
            pltpu.sync_copy(wbuf_v, out_ref.at[cid, pl.ds(off, WB)])

    def agg(table, srcm, dstm):
        k = pl.kernel(
            body,
            out_type=jax.ShapeDtypeStruct((NC, NPAD, d), jnp.float32),
            mesh=plsc.VectorSubcoreMesh(core_axis_name="c",
                                        subcore_axis_name="s"),
            scratch_types=[pltpu.VMEM((nstg, kc), jnp.int32),
                           pltpu.VMEM((nstg, kc), jnp.int32),
                           pltpu.VMEM((kc, d), jnp.float32),
                           pltpu.VMEM((kc, d), jnp.float32),
                           pltpu.VMEM((kc, d), jnp.float32),
                           pltpu.VMEM((kc, d), jnp.float32),
                           pltpu.VMEM((WB, d), jnp.float32),
                           pltpu.VMEM_SHARED((NPAD, d), jnp.float32)]
            + [pltpu.SemaphoreType.DMA] * 8,
            compiler_params=pltpu.CompilerParams(
                needs_layout_passes=False,
                use_tc_tiling_on_sc=(d % 128 == 0)),
        )
        return k(table, srcm, dstm)

    return agg


KC128 = 50
KC16 = 125
_agg128 = _make_agg(HID, KC128, 40)
_agg16 = _make_agg_ring(NCLS, KC16, 16)


def _mid_body(a0_ref, a1_ref, di_ref, do_ref, b0_ref, w1_ref, g_ref):
    s = a0_ref[...] + a1_ref[...]
    h0 = jnp.maximum(s * di_ref[...] + b0_ref[...], 0.0)
    g = jnp.dot(h0, w1_ref[...], preferred_element_type=jnp.float32,
                precision=_HIGH)
    g_ref[...] = g * do_ref[...]


def _mid(aggp, dif, dof, b0, w1):
    blk = 1024
    nblk = NPAD // blk
    ap2 = aggp.reshape(NC * NPAD, HID)
    return pl.pallas_call(
        _mid_body,
        grid=(nblk,),
        in_specs=[pl.BlockSpec((blk, HID), lambda i: (i, 0)),
                  pl.BlockSpec((blk, HID), lambda i: (nblk + i, 0)),
                  pl.BlockSpec((blk, 1), lambda i: (i, 0)),
                  pl.BlockSpec((blk, 1), lambda i: (i, 0)),
                  pl.BlockSpec((1, HID), lambda i: (0, 0)),
                  pl.BlockSpec((HID, NCLS), lambda i: (0, 0))],
        out_specs=pl.BlockSpec((blk, NCLS), lambda i: (i, 0)),
        out_shape=jax.ShapeDtypeStruct((NPAD, NCLS), jnp.float32),
    )(ap2, ap2, dif, dof, b0, w1)


PACK = 128 // NCLS
NROWS = NPAD // PACK


def _fin_body(ap_ref, dp_ref, b1_ref, o_ref):
    r = lax.broadcasted_iota(jnp.int32, (PACK, 128), 0)
    l = lax.broadcasted_iota(jnp.int32, (PACK, 128), 1)
    hot = jnp.where(l // NCLS == r, 1.0, 0.0).astype(jnp.float32)
    dp = jnp.dot(dp_ref[...], hot, preferred_element_type=jnp.float32)
    o_ref[...] = (ap_ref[0] + ap_ref[1]) * dp + b1_ref[...]


def _final(aggp2r, difp, b1p):
    blk = 128
    return pl.pallas_call(
        _fin_body,
        grid=(NROWS // blk,),
        in_specs=[pl.BlockSpec((NC, blk, 128), lambda i: (0, i, 0)),
                  pl.BlockSpec((blk, PACK), lambda i: (i, 0)),
                  pl.BlockSpec((1, 128), lambda i: (0, 0))],
        out_specs=pl.BlockSpec((blk, 128), lambda i: (i, 0)),
        out_shape=jax.ShapeDtypeStruct((NROWS, 128), jnp.float32),
    )(aggp2r, difp, b1p)


def kernel(x0, x1, params, edge_index):
    src = edge_index[0]
    dst = edge_index[1]
    srcm128 = src.reshape(E // KC128, KC128)
    dstm128 = dst.reshape(E // KC128, KC128)
    srcm16 = src.reshape(E // KC16, KC16)
    dstm16 = dst.reshape(E // KC16, KC16)

    hs, hd = _degrees(src, dst)
    dof, dif = _degfin(hs, hd)

    c0, s0, c1, s1 = _cov(x0, x1)
    g0, g1, gb = _fold(c0, s0, c1, s1, params)
    z = _z_kernel(x0, x1, g0, g1, gb, dof)

    aggp = _agg128(z, srcm128, dstm128)
    g = _mid(aggp, dif, dof, params['gcn_b0'][None, :], params['gcn_W1'])
    aggp2 = _agg16(g, srcm16, dstm16)
    aggp2r = aggp2.reshape(NC, NROWS, 128)
    difp = dif.reshape(NROWS, PACK)
    b1p = jnp.tile(params['gcn_b1'], PACK)[None, :]
    fin = _final(aggp2r, difp, b1p)
    return fin.reshape(NPAD, NCLS)[:N]
